# baseline (speedup 1.0000x reference)
import functools

import jax
import jax.numpy as jnp
from jax import lax
from jax.experimental import pallas as pl
from jax.experimental.pallas import tpu as pltpu
from jax.experimental.pallas import tpu_sc as plsc

NC = 2
NS = 16
NW = NC * NS
LANES = 16

D = 64
DM = D + 1
CHUNK = 128
GRPS = CHUNK // LANES
EPS = 1e-15


def _rsqrt(s):
    i = plsc.bitcast(s, jnp.int32)
    i = jnp.int32(0x5F3759DF) - lax.shift_right_arithmetic(i, 1)
    r = plsc.bitcast(i, jnp.float32)
    for _ in range(3):
        r = r * (1.5 - 0.5 * s * r * r)
    return r


_LOG1P_Q = (-0.023979573072231225, 0.10150004715402716, -0.21029369270420023,
            0.325295141401553, -0.4993725978465223, 0.9999918285309963)


def _softplus(c):
    u = jnp.exp(-jnp.abs(c))
    p = jnp.float32(_LOG1P_Q[0])
    for q in _LOG1P_Q[1:]:
        p = p * u + q
    return jnp.maximum(c, 0.0) + u * p


def _sc_body(n_lookups, x_hbm, mean_hbm, covar_hbm, outm_hbm, outc_hbm,
             idx_all, mrows, crows, outm_v, outc_v, coltab, scale_all,
             sem_gm, sem_gc, sem_om, sem_oc):
    per_w = n_lookups // NW
    n_chunks = per_w // CHUNK
    wid = lax.axis_index("s") * NC + lax.axis_index("c")
    base = wid * per_w
    iota = lax.iota(jnp.int32, LANES)

    pltpu.sync_copy(x_hbm.at[pl.ds(base, per_w)], idx_all)

    def fill_col(k, carry):
        col = iota + k
        coltab[pl.ds(k * LANES, LANES)] = jnp.where(col >= D, col - D, col)
        return carry

    lax.fori_loop(0, D, fill_col, 0)

    def start_gather(c, b):
        idxs = idx_all.at[pl.ds(c * CHUNK, CHUNK)]
        pltpu.async_copy(mean_hbm.at[idxs], mrows[b], sem_gm[b])
        pltpu.async_copy(covar_hbm.at[idxs], crows[b], sem_gc[b])

    def wait_gather(b):
        idxs = idx_all.at[pl.ds(0, CHUNK)]
        pltpu.make_async_copy(mean_hbm.at[idxs], mrows[b], sem_gm[b]).wait()
        pltpu.make_async_copy(covar_hbm.at[idxs], crows[b], sem_gc[b]).wait()

    def start_out(c, b):
        off = base + c * CHUNK
        pltpu.async_copy(outm_v[b], outm_hbm.at[pl.ds(off * DM, CHUNK * DM)],
                         sem_om[b])
        pltpu.async_copy(outc_v[b], outc_hbm.at[pl.ds(off * D, CHUNK * D)],
                         sem_oc[b])

    def wait_out(b):
        pltpu.make_async_copy(
            outm_v[b], outm_hbm.at[pl.ds(base * DM, CHUNK * DM)],
            sem_om[b]).wait()
        pltpu.make_async_copy(
            outc_v[b], outc_hbm.at[pl.ds(base * D, CHUNK * D)],
            sem_oc[b]).wait()

    def compute(mr, cr, om, oc):
        def grpa(gi, carry):
            rows = gi * LANES + iota
            accs = [jnp.zeros((LANES,), jnp.float32) for _ in range(4)]
            for k in range(D):
                col = coltab[pl.ds(k * LANES, LANES)]
                v = plsc.load_gather(mr, [rows, col])
                accs[k % 4] = accs[k % 4] + v * v
            ssum = (accs[0] + accs[1]) + (accs[2] + accs[3])
            s = jnp.maximum(ssum, EPS)
            r = _rsqrt(s)
            n = s * r
            e = jnp.exp(n)
            ei = jnp.exp(-n)
            cosh = 0.5 * (e + ei)
            scale = (0.5 * (e - ei)) * r
            scale_all[pl.ds(gi * LANES, LANES)] = scale
            plsc.store_scatter(om, [rows * DM], cosh)
            return carry

        lax.fori_loop(0, GRPS, grpa, 0)

        def lkb(lrow, carry):
            splat = plsc.load_gather(
                scale_all, [jnp.full((LANES,), lrow, jnp.int32)])
            obase = lrow * DM + 1
            for j in range(4):
                m = mr[lrow, pl.ds(j * LANES, LANES)]
                plsc.store_scatter(om, [obase + j * LANES + iota], m * splat)
                cv = cr[lrow, pl.ds(j * LANES, LANES)]
                oc[pl.ds(lrow * D + j * LANES, LANES)] = _softplus(cv)
            return carry

        lax.fori_loop(0, CHUNK, lkb, 0, unroll=2)

    start_gather(0, 0)
    start_gather(1, 1)

    def pair_body(g, carry):
        for b in range(2):
            c = 2 * g + b
            wait_gather(b)

            @pl.when(c >= 2)
            def _():
                wait_out(b)

            compute(mrows[b], crows[b], outm_v[b], outc_v[b])
            start_out(c, b)

            @pl.when(c + 2 < n_chunks)
            def _():
                start_gather(c + 2, b)
        return carry

    lax.fori_loop(0, n_chunks // 2, pair_body, 0)
    wait_out(0)
    wait_out(1)


@functools.partial(jax.jit, static_argnames=("n_lookups",))
def _run(x_flat, mean_table, covar_table, n_lookups):
    per_w = n_lookups // NW
    mesh = plsc.VectorSubcoreMesh(
        core_axis_name="c", subcore_axis_name="s",
        num_cores=NC, num_subcores=NS)
    fn = pl.kernel(
        functools.partial(_sc_body, n_lookups),
        out_type=(
            jax.ShapeDtypeStruct((n_lookups * DM,), jnp.float32),
            jax.ShapeDtypeStruct((n_lookups * D,), jnp.float32),
        ),
        mesh=mesh,
        compiler_params=pltpu.CompilerParams(
            needs_layout_passes=False, use_tc_tiling_on_sc=False),
        scratch_types=[
            pltpu.VMEM((per_w,), jnp.int32),
            [pltpu.VMEM((CHUNK, D), jnp.float32)] * 2,
            [pltpu.VMEM((CHUNK, D), jnp.float32)] * 2,
            [pltpu.VMEM((CHUNK * DM,), jnp.float32)] * 2,
            [pltpu.VMEM((CHUNK * D,), jnp.float32)] * 2,
            pltpu.VMEM((D * LANES,), jnp.int32),
            pltpu.VMEM((CHUNK,), jnp.float32),
            [pltpu.SemaphoreType.DMA] * 2,
            [pltpu.SemaphoreType.DMA] * 2,
            [pltpu.SemaphoreType.DMA] * 2,
            [pltpu.SemaphoreType.DMA] * 2,
        ],
    )
    return fn(x_flat, mean_table, covar_table)


def kernel(x, mean_table, covar_table):
    b, l = x.shape
    n = b * l
    outm, outc = _run(x.reshape(n), mean_table, covar_table, n)
    return outm.reshape(b, l, DM), outc.reshape(b, l, D)


# speedup vs baseline: 2.4908x; 2.4908x over previous
import functools

import jax
import jax.numpy as jnp
from jax import lax
from jax.experimental import pallas as pl
from jax.experimental.pallas import tpu as pltpu
from jax.experimental.pallas import tpu_sc as plsc

NC = 2
NS = 16
NW = NC * NS
LANES = 16

D = 64
DM = D + 1
CHUNK = 128
GRPS = CHUNK // LANES
EPS = 1e-15


def _rsqrt(s):
    i = plsc.bitcast(s, jnp.int32)
    i = jnp.int32(0x5F3759DF) - lax.shift_right_arithmetic(i, 1)
    r = plsc.bitcast(i, jnp.float32)
    for _ in range(3):
        r = r * (1.5 - 0.5 * s * r * r)
    return r


_LOG1P_Q = (-0.023979573072231225, 0.10150004715402716, -0.21029369270420023,
            0.325295141401553, -0.4993725978465223, 0.9999918285309963)


def _softplus(c):
    u = jnp.exp(-jnp.abs(c))
    p = jnp.float32(_LOG1P_Q[0])
    for q in _LOG1P_Q[1:]:
        p = p * u + q
    return jnp.maximum(c, 0.0) + u * p


def _sc_body(n_lookups, x_hbm, mean_hbm, covar_hbm, outm_hbm, outc_hbm,
             idx_all, mrows, crows, outm_v, outc_v, coltab, scale_all,
             sem_gm, sem_gc, sem_om, sem_oc):
    per_w = n_lookups // NW
    n_chunks = per_w // CHUNK
    wid = lax.axis_index("s") * NC + lax.axis_index("c")
    base = wid * per_w
    iota = lax.iota(jnp.int32, LANES)

    pltpu.sync_copy(x_hbm.at[pl.ds(base, per_w)], idx_all)

    def fill_col(k, carry):
        col = iota + k
        coltab[pl.ds(k * LANES, LANES)] = jnp.where(col >= D, col - D, col)
        return carry

    lax.fori_loop(0, D, fill_col, 0)

    def start_gather(c, b):
        idxs = idx_all.at[pl.ds(c * CHUNK, CHUNK)]
        pltpu.async_copy(mean_hbm.at[idxs], mrows[b], sem_gm[b])
        pltpu.async_copy(covar_hbm.at[idxs], crows[b], sem_gc[b])

    def wait_gather(b):
        idxs = idx_all.at[pl.ds(0, CHUNK)]
        pltpu.make_async_copy(mean_hbm.at[idxs], mrows[b], sem_gm[b]).wait()
        pltpu.make_async_copy(covar_hbm.at[idxs], crows[b], sem_gc[b]).wait()

    def start_out(c, b):
        off = base + c * CHUNK
        pltpu.async_copy(outm_v[b], outm_hbm.at[pl.ds(off * DM, CHUNK * DM)],
                         sem_om[b])
        pltpu.async_copy(outc_v[b], outc_hbm.at[pl.ds(off * D, CHUNK * D)],
                         sem_oc[b])

    def wait_out(b):
        pltpu.make_async_copy(
            outm_v[b], outm_hbm.at[pl.ds(base * DM, CHUNK * DM)],
            sem_om[b]).wait()
        pltpu.make_async_copy(
            outc_v[b], outc_hbm.at[pl.ds(base * D, CHUNK * D)],
            sem_oc[b]).wait()

    def compute(mr, cr, om, oc):
        @functools.partial(plsc.parallel_loop, 0, GRPS, unroll=2)
        def grpa(gi):
            rows = gi * LANES + iota
            accs = [jnp.zeros((LANES,), jnp.float32) for _ in range(4)]
            for k in range(D):
                col = coltab[pl.ds(k * LANES, LANES)]
                v = plsc.load_gather(mr, [rows, col])
                accs[k % 4] = accs[k % 4] + v * v
            ssum = (accs[0] + accs[1]) + (accs[2] + accs[3])
            s = jnp.maximum(ssum, EPS)
            r = _rsqrt(s)
            n = s * r
            e = jnp.exp(n)
            ei = jnp.exp(-n)
            cosh = 0.5 * (e + ei)
            scale = (0.5 * (e - ei)) * r
            scale_all[pl.ds(gi * LANES, LANES)] = scale
            plsc.store_scatter(om, [rows * DM], cosh)

        @functools.partial(plsc.parallel_loop, 0, CHUNK, unroll=2)
        def lkb(lrow):
            splat = plsc.load_gather(
                scale_all, [jnp.full((LANES,), lrow, jnp.int32)])
            obase = lrow * DM + 1
            for j in range(4):
                m = mr[lrow, pl.ds(j * LANES, LANES)]
                plsc.store_scatter(om, [obase + j * LANES + iota], m * splat)
                cv = cr[lrow, pl.ds(j * LANES, LANES)]
                oc[pl.ds(lrow * D + j * LANES, LANES)] = _softplus(cv)

    start_gather(0, 0)
    start_gather(1, 1)

    def pair_body(g, carry):
        for b in range(2):
            c = 2 * g + b
            wait_gather(b)

            @pl.when(c >= 2)
            def _():
                wait_out(b)

            compute(mrows[b], crows[b], outm_v[b], outc_v[b])
            start_out(c, b)

            @pl.when(c + 2 < n_chunks)
            def _():
                start_gather(c + 2, b)
        return carry

    lax.fori_loop(0, n_chunks // 2, pair_body, 0)
    wait_out(0)
    wait_out(1)


@functools.partial(jax.jit, static_argnames=("n_lookups",))
def _run(x_flat, mean_table, covar_table, n_lookups):
    per_w = n_lookups // NW
    mesh = plsc.VectorSubcoreMesh(
        core_axis_name="c", subcore_axis_name="s",
        num_cores=NC, num_subcores=NS)
    fn = pl.kernel(
        functools.partial(_sc_body, n_lookups),
        out_type=(
            jax.ShapeDtypeStruct((n_lookups * DM,), jnp.float32),
            jax.ShapeDtypeStruct((n_lookups * D,), jnp.float32),
        ),
        mesh=mesh,
        compiler_params=pltpu.CompilerParams(
            needs_layout_passes=False, use_tc_tiling_on_sc=False),
        scratch_types=[
            pltpu.VMEM((per_w,), jnp.int32),
            [pltpu.VMEM((CHUNK, D), jnp.float32)] * 2,
            [pltpu.VMEM((CHUNK, D), jnp.float32)] * 2,
            [pltpu.VMEM((CHUNK * DM,), jnp.float32)] * 2,
            [pltpu.VMEM((CHUNK * D,), jnp.float32)] * 2,
            pltpu.VMEM((D * LANES,), jnp.int32),
            pltpu.VMEM((CHUNK,), jnp.float32),
            [pltpu.SemaphoreType.DMA] * 2,
            [pltpu.SemaphoreType.DMA] * 2,
            [pltpu.SemaphoreType.DMA] * 2,
            [pltpu.SemaphoreType.DMA] * 2,
        ],
    )
    return fn(x_flat, mean_table, covar_table)


def kernel(x, mean_table, covar_table):
    b, l = x.shape
    n = b * l
    outm, outc = _run(x.reshape(n), mean_table, covar_table, n)
    return outm.reshape(b, l, DM), outc.reshape(b, l, D)
